# baseline (speedup 1.0000x reference)
import functools

import jax
import jax.numpy as jnp
from jax import lax
from jax.experimental import pallas as pl
from jax.experimental.pallas import tpu as pltpu
from jax.experimental.pallas import tpu_sc as plsc

_LANES = 16


@functools.cache
def _build_sampler(B, V, D):
    info = plsc.get_sparse_core_info()
    nc, ns = info.num_cores, info.num_subcores
    nw = nc * ns
    assert B % (8 * nw) == 0 and D % _LANES == 0
    b_per_w = B // nw
    mesh = plsc.VectorSubcoreMesh(core_axis_name="c", subcore_axis_name="s")

    @functools.partial(
        pl.kernel,
        mesh=mesh,
        out_type=jax.ShapeDtypeStruct((B, D), jnp.float32),
        compiler_params=pltpu.CompilerParams(use_tc_tiling_on_sc=False),
        scratch_types=[
            pltpu.VMEM((b_per_w,), jnp.int32),
            pltpu.VMEM((b_per_w, D), jnp.float32),
            pltpu.VMEM((b_per_w, D), jnp.float32),
            pltpu.VMEM((b_per_w, D), jnp.float32),
            pltpu.SemaphoreType.DMA,
            pltpu.SemaphoreType.DMA,
        ],
    )
    def sampler(y_hbm, mu_hbm, sigma_hbm, eps_hbm, out_hbm,
                idx_v, mu_v, sg_v, ep_v, sem_mu, sem_sg):
        wid = lax.axis_index("s") * nc + lax.axis_index("c")
        base = wid * b_per_w
        pltpu.sync_copy(y_hbm.at[pl.ds(base, b_per_w)], idx_v)
        cp_mu = pltpu.async_copy(mu_hbm.at[idx_v], mu_v, sem_mu)
        cp_sg = pltpu.async_copy(sigma_hbm.at[idx_v], sg_v, sem_sg)
        pltpu.sync_copy(eps_hbm.at[pl.ds(base, b_per_w)], ep_v)
        cp_mu.wait()
        cp_sg.wait()

        def row(i, carry):
            for c in range(D // _LANES):
                sl = pl.ds(c * _LANES, _LANES)
                mu_v[i, sl] = mu_v[i, sl] + sg_v[i, sl] * ep_v[i, sl]
            return carry

        lax.fori_loop(0, b_per_w, row, 0, unroll=2)
        pltpu.sync_copy(mu_v, out_hbm.at[pl.ds(base, b_per_w)])

    return sampler


def kernel(y, mu, sigma):
    B = y.shape[0]
    V, D = mu.shape
    eps = jax.random.normal(jax.random.key(42), (B, D), dtype=mu.dtype)
    return _build_sampler(B, V, D)(y, mu, sigma, eps)
